# Initial kernel scaffold; baseline (speedup 1.0000x reference)
import jax
import jax.numpy as jnp
from jax.experimental import pallas as pl


def kernel(y_pred, y_gt, freqs, ranges):
    raise NotImplementedError("write your pallas kernel here")


# speedup vs baseline: 2.6632x; 2.6632x over previous
import functools

import jax
import jax.numpy as jnp
from jax import lax
from jax.experimental import pallas as pl
from jax.experimental.pallas import tpu as pltpu

HIST_LEN = 10
ALPHA = 1.0
EPSILON = 0.1
GAMMA = 1.0

ROWS, COLS = 1024, 4096
N_ELEMS = ROWS * COLS
BLOCK_ROWS = 256
GRID = ROWS // BLOCK_ROWS


def _exact_thresholds(ranges):
    rk = ranges[:9]
    rk1 = ranges[1:]
    lo = lax.bitcast_convert_type(rk, jnp.int32)
    hi = lax.bitcast_convert_type(rk1, jnp.int32)

    def body(_, lohi):
        lo, hi = lohi
        mid = (lo + hi) // 2
        g = lax.bitcast_convert_type(mid, jnp.float32)
        q = jnp.abs(g - rk1) < jnp.abs(g - rk)
        return jnp.where(q, lo, mid), jnp.where(q, mid, hi)

    lo, hi = lax.fori_loop(0, 32, body, (lo, hi))
    return lax.bitcast_convert_type(hi, jnp.float32)


def _tc_body(scal_ref, pred_ref, gt_ref, ow_ref, owd2_ref):
    @pl.when(pl.program_id(0) == 0)
    def _():
        ow_ref[0, 0] = 0.0
        owd2_ref[0, 0] = 0.0

    g = gt_ref[...]
    p = pred_ref[...]
    d = g - p
    d2 = d * d
    w = jnp.full_like(g, scal_ref[1, 15])
    for k in range(9):
        w = w + jnp.where(g >= scal_ref[0, k], scal_ref[1, k], 0.0)
    ow_ref[0, 0] += jnp.sum(w)
    owd2_ref[0, 0] += jnp.sum(w * d2)


def kernel(y_pred, y_gt, freqs, ranges):
    ranges = ranges.astype(jnp.float32)
    t = _exact_thresholds(ranges)
    fsum = jnp.sum(freqs).astype(jnp.float32)
    dens = freqs.astype(jnp.float32) / fsum
    wtab = jnp.maximum(1.0 - ALPHA * dens, EPSILON)
    deltas = wtab[1:] - wtab[:9]

    scal = jnp.zeros((2, 16), jnp.float32)
    scal = scal.at[0, :9].set(t).at[0, 9:].set(9e9)
    scal = scal.at[1, :9].set(deltas).at[1, 15].set(wtab[0])

    ow, owd2 = pl.pallas_call(
        _tc_body,
        grid=(GRID,),
        in_specs=[
            pl.BlockSpec(memory_space=pltpu.SMEM),
            pl.BlockSpec((BLOCK_ROWS, COLS), lambda i: (i, 0)),
            pl.BlockSpec((BLOCK_ROWS, COLS), lambda i: (i, 0)),
        ],
        out_specs=[
            pl.BlockSpec(memory_space=pltpu.SMEM),
            pl.BlockSpec(memory_space=pltpu.SMEM),
        ],
        out_shape=[
            jax.ShapeDtypeStruct((1, 1), jnp.float32),
            jax.ShapeDtypeStruct((1, 1), jnp.float32),
        ],
    )(scal, y_pred, y_gt)

    a = ow[0, 0]
    b = owd2[0, 0]
    return GAMMA * jnp.float32(N_ELEMS) * b / a
